# baseline (speedup 1.0000x reference)
import functools

import jax
import jax.numpy as jnp
from jax import lax
from jax.experimental import pallas as pl
from jax.experimental.pallas import tpu as pltpu
from jax.experimental.pallas import tpu_sc as plsc

PAD_IDX = 1
MASK_ID = 32
LN_EPS = 1e-05
B, S, D = 4, 2048, 2048
VOCAB, MAX_POS = 33, 4096

NC, NS = 2, 16
NW = NC * NS
TOK = B * S
TPW = TOK // NW
CHUNK = 8
NCHUNK = TPW // CHUNK
NVREG = D // 16


def _prep_body(ids_ref, attn_ref, pos_ref, wsc_ref):
    ids = ids_ref[...]
    attn = attn_ref[...]
    nonpad = (ids != PAD_IDX).astype(jnp.int32)
    c = nonpad
    sh = 1
    while sh < S:
        c = c + jnp.concatenate(
            [jnp.zeros((B, sh), jnp.int32), c[:, : S - sh]], axis=1)
        sh *= 2
    pos_ref[...] = c * nonpad + PAD_IDX
    is_mask = ids == MASK_ID
    n_mask = jnp.sum(is_mask.astype(jnp.float32), axis=1, keepdims=True)
    src = jnp.sum(attn, axis=1, keepdims=True)
    scale = (1.0 - 0.15 * 0.8) / (1.0 - n_mask / src)
    wsc_ref[...] = jnp.where(is_mask, 0.0, jnp.broadcast_to(scale, (B, S)))


def _prep(ids, attn):
    return pl.pallas_call(
        _prep_body,
        out_shape=[
            jax.ShapeDtypeStruct((B, S), jnp.int32),
            jax.ShapeDtypeStruct((B, S), jnp.float32),
        ],
    )(ids, attn)


def _sc_body(ids_hbm, pos2_hbm, wsc_hbm, am_hbm, wtab_hbm, ptab_hbm,
             g_hbm, b_hbm, out_hbm,
             wtab_v, prows_v, obuf_v, g_v, b_v,
             ids_v, pos_v, wsc_v, am_v, sem):
    wid = lax.axis_index("s") * NC + lax.axis_index("c")
    base = wid * TPW

    pltpu.sync_copy(ids_hbm.at[pl.ds(base, TPW)], ids_v)
    pltpu.sync_copy(pos2_hbm.at[pl.ds(wid * NCHUNK, NCHUNK)], pos_v)
    pltpu.sync_copy(wsc_hbm.at[pl.ds(base, TPW)], wsc_v)
    pltpu.sync_copy(am_hbm.at[pl.ds(base, TPW)], am_v)
    pltpu.sync_copy(wtab_hbm, wtab_v)
    pltpu.sync_copy(g_hbm, g_v)
    pltpu.sync_copy(b_hbm, b_v)

    iota16 = lax.iota(jnp.int32, 16)
    zeros16 = jnp.zeros((16,), jnp.int32)
    inv_d = 1.0 / D

    def chunk_body(c, carry):
        pltpu.async_copy(ptab_hbm.at[pos_v.at[c]], prows_v, sem).wait()

        for t in range(CHUNK):
            tok = c * CHUNK + t
            tok_splat = zeros16 + tok
            row_splat = plsc.load_gather(ids_v, [tok_splat])
            wscv = plsc.load_gather(wsc_v, [tok_splat])
            amv = plsc.load_gather(am_v, [tok_splat])

            def p1(j, sc):
                s, ss = sc
                cols = iota16 + j * 16
                w = plsc.load_gather(wtab_v, [row_splat, cols])
                p = prows_v[t, pl.ds(j * 16, 16)]
                x = w * wscv + p
                obuf_v[t, pl.ds(j * 16, 16)] = x
                return s + x, ss + x * x

            s, ss = lax.fori_loop(
                0, NVREG, p1,
                (jnp.zeros((16,), jnp.float32), jnp.zeros((16,), jnp.float32)))
            mu = jnp.sum(s) * inv_d
            var = jnp.sum(ss) * inv_d - mu * mu
            vv = jnp.broadcast_to(var + LN_EPS, (16,))
            yi = jnp.int32(0x5F3759DF) - (
                plsc.bitcast(vv, jnp.int32) >> jnp.int32(1))
            y = plsc.bitcast(yi, jnp.float32)
            for _ in range(3):
                y = y * (1.5 - 0.5 * vv * y * y)
            a1 = y * amv
            a0 = (-mu) * a1

            def p2(j, _):
                sl = pl.ds(j * 16, 16)
                x = obuf_v[t, sl]
                obuf_v[t, sl] = g_v[sl] * (x * a1 + a0) + b_v[sl] * amv
                return 0

            lax.fori_loop(0, NVREG, p2, 0)

        pltpu.sync_copy(obuf_v, out_hbm.at[pl.ds(base + c * CHUNK, CHUNK)])
        return carry

    lax.fori_loop(0, NCHUNK, chunk_body, 0)


@functools.partial(jax.jit, static_argnums=())
def _sc_embed(ids_f, pos2, wsc_f, am_f, word_emb, pos_emb, g, b):
    mesh = plsc.VectorSubcoreMesh(core_axis_name="c", subcore_axis_name="s")
    k = functools.partial(
        pl.kernel,
        mesh=mesh,
        compiler_params=pltpu.CompilerParams(needs_layout_passes=False),
        out_type=jax.ShapeDtypeStruct((TOK, D), jnp.float32),
        scratch_types=[
            pltpu.VMEM((VOCAB, D), jnp.float32),
            pltpu.VMEM((CHUNK, D), jnp.float32),
            pltpu.VMEM((CHUNK, D), jnp.float32),
            pltpu.VMEM((D,), jnp.float32),
            pltpu.VMEM((D,), jnp.float32),
            pltpu.VMEM((TPW,), jnp.int32),
            pltpu.VMEM((NCHUNK, CHUNK), jnp.int32),
            pltpu.VMEM((TPW,), jnp.float32),
            pltpu.VMEM((TPW,), jnp.float32),
            pltpu.SemaphoreType.DMA,
        ],
    )(_sc_body)
    return k(ids_f, pos2, wsc_f, am_f, word_emb, pos_emb, g, b)


def kernel(input_ids, attention_mask, word_emb, pos_emb, ln_gamma, ln_beta):
    ids = input_ids.astype(jnp.int32)
    attn = attention_mask.astype(jnp.float32)
    pos_ids, wscale = _prep(ids, attn)
    out = _sc_embed(
        ids.reshape(TOK), pos_ids.reshape(NW * NCHUNK, CHUNK),
        wscale.reshape(TOK), attn.reshape(TOK),
        word_emb, pos_emb, ln_gamma, ln_beta)
    return out.reshape(B, S, D)


# speedup vs baseline: 1.4744x; 1.4744x over previous
import functools

import jax
import jax.numpy as jnp
from jax import lax
from jax.experimental import pallas as pl
from jax.experimental.pallas import tpu as pltpu
from jax.experimental.pallas import tpu_sc as plsc

PAD_IDX = 1
MASK_ID = 32
LN_EPS = 1e-05
B, S, D = 4, 2048, 2048
VOCAB, MAX_POS = 33, 4096

NC, NS = 2, 16
NW = NC * NS
TOK = B * S
TPW = TOK // NW
CHUNK = 8
NCHUNK = TPW // CHUNK
NVREG = D // 16


def _prep_body(ids_ref, attn_ref, pos_ref, wsc_ref):
    ids = ids_ref[...]
    attn = attn_ref[...]
    nonpad = (ids != PAD_IDX).astype(jnp.int32)
    c = nonpad
    sh = 1
    while sh < S:
        c = c + jnp.concatenate(
            [jnp.zeros((B, sh), jnp.int32), c[:, : S - sh]], axis=1)
        sh *= 2
    pos_ref[...] = c * nonpad + PAD_IDX
    is_mask = ids == MASK_ID
    n_mask = jnp.sum(is_mask.astype(jnp.float32), axis=1, keepdims=True)
    src = jnp.sum(attn, axis=1, keepdims=True)
    scale = (1.0 - 0.15 * 0.8) / (1.0 - n_mask / src)
    wsc_ref[...] = jnp.where(is_mask, 0.0, jnp.broadcast_to(scale, (B, S)))


def _prep(ids, attn):
    return pl.pallas_call(
        _prep_body,
        out_shape=[
            jax.ShapeDtypeStruct((B, S), jnp.int32),
            jax.ShapeDtypeStruct((B, S), jnp.float32),
        ],
    )(ids, attn)


def _sc_body(ids_hbm, pos2_hbm, wsc_hbm, am_hbm, wtab_hbm, ptab_hbm,
             g_hbm, b_hbm, out_hbm,
             wtab_v, prows_v, g_v, b_v,
             ids_v, pos_v, wsc_v, am_v, sem):
    wid = lax.axis_index("s") * NC + lax.axis_index("c")
    base = wid * TPW

    pltpu.sync_copy(ids_hbm.at[pl.ds(base, TPW)], ids_v)
    pltpu.sync_copy(pos2_hbm.at[pl.ds(wid * NCHUNK, NCHUNK)], pos_v)
    pltpu.sync_copy(wsc_hbm.at[pl.ds(base, TPW)], wsc_v)
    pltpu.sync_copy(am_hbm.at[pl.ds(base, TPW)], am_v)
    pltpu.sync_copy(wtab_hbm, wtab_v)
    pltpu.sync_copy(g_hbm, g_v)
    pltpu.sync_copy(b_hbm, b_v)

    iota16 = lax.iota(jnp.int32, 16)
    zeros16 = jnp.zeros((16,), jnp.int32)
    inv_d = 1.0 / D

    zf = jnp.zeros((16,), jnp.float32)

    def chunk_body(c, carry):
        pltpu.async_copy(ptab_hbm.at[pos_v.at[c]], prows_v, sem).wait()
        tok0 = c * CHUNK

        row_splats, wscvs, amvs = [], [], []
        for t in range(CHUNK):
            tok_splat = zeros16 + (tok0 + t)
            row_splats.append(plsc.load_gather(ids_v, [tok_splat]))
            wscvs.append(plsc.load_gather(wsc_v, [tok_splat]))
            amvs.append(plsc.load_gather(am_v, [tok_splat]))

        def p1(j, acc):
            cols = iota16 + j * 16
            sl = pl.ds(j * 16, 16)
            new = []
            for t in range(CHUNK):
                w = plsc.load_gather(wtab_v, [row_splats[t], cols])
                x = w * wscvs[t] + prows_v[t, sl]
                prows_v[t, sl] = x
                new.append(acc[2 * t] + x)
                new.append(acc[2 * t + 1] + x * x)
            return tuple(new)

        acc = lax.fori_loop(0, NVREG, p1, (zf,) * (2 * CHUNK))

        a1s, a0s = [], []
        for t in range(CHUNK):
            mu = jnp.sum(acc[2 * t]) * inv_d
            var = jnp.sum(acc[2 * t + 1]) * inv_d - mu * mu
            vv = jnp.broadcast_to(var + LN_EPS, (16,))
            yi = jnp.int32(0x5F3759DF) - (
                plsc.bitcast(vv, jnp.int32) >> jnp.int32(1))
            y = plsc.bitcast(yi, jnp.float32)
            for _ in range(3):
                y = y * (1.5 - 0.5 * vv * y * y)
            a1s.append(y * amvs[t])
            a0s.append((-mu) * y * amvs[t])

        def p2(j, carry2):
            sl = pl.ds(j * 16, 16)
            g = g_v[sl]
            b = b_v[sl]
            for t in range(CHUNK):
                x = prows_v[t, sl]
                prows_v[t, sl] = g * (x * a1s[t] + a0s[t]) + b * amvs[t]
            return carry2

        lax.fori_loop(0, NVREG, p2, 0)

        pltpu.sync_copy(prows_v, out_hbm.at[pl.ds(base + tok0, CHUNK)])
        return carry

    lax.fori_loop(0, NCHUNK, chunk_body, 0)


@functools.partial(jax.jit, static_argnums=())
def _sc_embed(ids_f, pos2, wsc_f, am_f, word_emb, pos_emb, g, b):
    mesh = plsc.VectorSubcoreMesh(core_axis_name="c", subcore_axis_name="s")
    k = functools.partial(
        pl.kernel,
        mesh=mesh,
        compiler_params=pltpu.CompilerParams(needs_layout_passes=False),
        out_type=jax.ShapeDtypeStruct((TOK, D), jnp.float32),
        scratch_types=[
            pltpu.VMEM((VOCAB, D), jnp.float32),
            pltpu.VMEM((CHUNK, D), jnp.float32),
            pltpu.VMEM((D,), jnp.float32),
            pltpu.VMEM((D,), jnp.float32),
            pltpu.VMEM((TPW,), jnp.int32),
            pltpu.VMEM((NCHUNK, CHUNK), jnp.int32),
            pltpu.VMEM((TPW,), jnp.float32),
            pltpu.VMEM((TPW,), jnp.float32),
            pltpu.SemaphoreType.DMA,
        ],
    )(_sc_body)
    return k(ids_f, pos2, wsc_f, am_f, word_emb, pos_emb, g, b)


def kernel(input_ids, attention_mask, word_emb, pos_emb, ln_gamma, ln_beta):
    ids = input_ids.astype(jnp.int32)
    attn = attention_mask.astype(jnp.float32)
    pos_ids, wscale = _prep(ids, attn)
    out = _sc_embed(
        ids.reshape(TOK), pos_ids.reshape(NW * NCHUNK, CHUNK),
        wscale.reshape(TOK), attn.reshape(TOK),
        word_emb, pos_emb, ln_gamma, ln_beta)
    return out.reshape(B, S, D)


# speedup vs baseline: 1.8159x; 1.2317x over previous
import functools

import jax
import jax.numpy as jnp
from jax import lax
from jax.experimental import pallas as pl
from jax.experimental.pallas import tpu as pltpu
from jax.experimental.pallas import tpu_sc as plsc

PAD_IDX = 1
MASK_ID = 32
LN_EPS = 1e-05
B, S, D = 4, 2048, 2048
VOCAB, MAX_POS = 33, 4096

NC, NS = 2, 16
NW = NC * NS
TOK = B * S
TPW = TOK // NW
CHUNK = 4
NCHUNK = TPW // CHUNK
NVREG = D // 16


def _prep_body(ids_ref, attn_ref, pos_ref, wsc_ref):
    ids = ids_ref[...]
    attn = attn_ref[...]
    nonpad = (ids != PAD_IDX).astype(jnp.int32)
    c = nonpad
    sh = 1
    while sh < S:
        c = c + jnp.concatenate(
            [jnp.zeros((B, sh), jnp.int32), c[:, : S - sh]], axis=1)
        sh *= 2
    pos_ref[...] = c * nonpad + PAD_IDX
    is_mask = ids == MASK_ID
    n_mask = jnp.sum(is_mask.astype(jnp.float32), axis=1, keepdims=True)
    src = jnp.sum(attn, axis=1, keepdims=True)
    scale = (1.0 - 0.15 * 0.8) / (1.0 - n_mask / src)
    wsc_ref[...] = jnp.where(is_mask, 0.0, jnp.broadcast_to(scale, (B, S)))


def _prep(ids, attn):
    return pl.pallas_call(
        _prep_body,
        out_shape=[
            jax.ShapeDtypeStruct((B, S), jnp.int32),
            jax.ShapeDtypeStruct((B, S), jnp.float32),
        ],
    )(ids, attn)


def _sc_body(ids_hbm, pos2_hbm, wsc_hbm, am_hbm, wtab_hbm, ptab_hbm,
             g_hbm, b_hbm, out_hbm,
             wtab_v, pr0_v, pr1_v, pr2_v, g_v, b_v,
             ids_v, pos_v, wsc_v, am_v,
             gs0, gs1, gs2, ws0, ws1, ws2):
    wid = lax.axis_index("s") * NC + lax.axis_index("c")
    base = wid * TPW

    pltpu.sync_copy(ids_hbm.at[pl.ds(base, TPW)], ids_v)
    pltpu.sync_copy(pos2_hbm.at[pl.ds(wid * NCHUNK, NCHUNK)], pos_v)
    pltpu.sync_copy(wsc_hbm.at[pl.ds(base, TPW)], wsc_v)
    pltpu.sync_copy(am_hbm.at[pl.ds(base, TPW)], am_v)
    pltpu.sync_copy(wtab_hbm, wtab_v)
    pltpu.sync_copy(g_hbm, g_v)
    pltpu.sync_copy(b_hbm, b_v)

    iota16 = lax.iota(jnp.int32, 16)
    zeros16 = jnp.zeros((16,), jnp.int32)
    inv_d = 1.0 / D

    zf = jnp.zeros((16,), jnp.float32)
    bufs = (pr0_v, pr1_v, pr2_v)
    gsems = (gs0, gs1, gs2)
    wsems = (ws0, ws1, ws2)

    def gather_start(c, i):
        pltpu.make_async_copy(ptab_hbm.at[pos_v.at[c]], bufs[i],
                              gsems[i]).start()

    def gather_wait(c, i):
        pltpu.make_async_copy(ptab_hbm.at[pos_v.at[c]], bufs[i],
                              gsems[i]).wait()

    def write_start(c, i):
        pltpu.make_async_copy(bufs[i],
                              out_hbm.at[pl.ds(base + c * CHUNK, CHUNK)],
                              wsems[i]).start()

    def write_wait(c, i):
        pltpu.make_async_copy(bufs[i],
                              out_hbm.at[pl.ds(base + c * CHUNK, CHUNK)],
                              wsems[i]).wait()

    def compute_chunk(c, i, issue_next):
        prows_v = bufs[i]
        gather_wait(c, i)
        tok0 = c * CHUNK

        row_splats, wscvs, amvs = [], [], []
        for t in range(CHUNK):
            tok_splat = zeros16 + (tok0 + t)
            row_splats.append(plsc.load_gather(ids_v, [tok_splat]))
            wscvs.append(plsc.load_gather(wsc_v, [tok_splat]))
            amvs.append(plsc.load_gather(am_v, [tok_splat]))

        def p1(j, acc):
            cols = iota16 + j * 16
            sl = pl.ds(j * 16, 16)
            new = []
            for t in range(CHUNK):
                w = plsc.load_gather(wtab_v, [row_splats[t], cols])
                x = w * wscvs[t] + prows_v[t, sl]
                prows_v[t, sl] = x
                new.append(acc[2 * t] + x)
                new.append(acc[2 * t + 1] + x * x)
            return tuple(new)

        acc = lax.fori_loop(0, NVREG, p1, (zf,) * (2 * CHUNK))

        a1s, a0s = [], []
        for t in range(CHUNK):
            mu = jnp.sum(acc[2 * t]) * inv_d
            var = jnp.sum(acc[2 * t + 1]) * inv_d - mu * mu
            vv = jnp.broadcast_to(var + LN_EPS, (16,))
            yi = jnp.int32(0x5F3759DF) - (
                plsc.bitcast(vv, jnp.int32) >> jnp.int32(1))
            y = plsc.bitcast(yi, jnp.float32)
            for _ in range(3):
                y = y * (1.5 - 0.5 * vv * y * y)
            a1s.append(y * amvs[t])
            a0s.append((-mu) * y * amvs[t])

        def p2(j, carry2):
            sl = pl.ds(j * 16, 16)
            g = g_v[sl]
            b = b_v[sl]
            for t in range(CHUNK):
                x = prows_v[t, sl]
                prows_v[t, sl] = g * (x * a1s[t] + a0s[t]) + b * amvs[t]
            return carry2

        lax.fori_loop(0, NVREG, p2, 0)

        if issue_next:
            nb = (i + 2) % 3

            @pl.when(c >= 1)
            def _():
                write_wait(c - 1, nb)

            gather_start(c + 2, nb)
        write_start(c, i)

    gather_start(0, 0)
    gather_start(1, 1)

    def ring_body(g, carry):
        c0 = g * 3
        compute_chunk(c0, 0, True)
        compute_chunk(c0 + 1, 1, True)
        compute_chunk(c0 + 2, 2, True)
        return carry

    lax.fori_loop(0, (NCHUNK - 4) // 3, ring_body, 0)

    compute_chunk(NCHUNK - 4, (NCHUNK - 4) % 3, True)
    compute_chunk(NCHUNK - 3, (NCHUNK - 3) % 3, True)
    compute_chunk(NCHUNK - 2, (NCHUNK - 2) % 3, False)
    compute_chunk(NCHUNK - 1, (NCHUNK - 1) % 3, False)
    write_wait(NCHUNK - 3, (NCHUNK - 3) % 3)
    write_wait(NCHUNK - 2, (NCHUNK - 2) % 3)
    write_wait(NCHUNK - 1, (NCHUNK - 1) % 3)


@functools.partial(jax.jit, static_argnums=())
def _sc_embed(ids_f, pos2, wsc_f, am_f, word_emb, pos_emb, g, b):
    mesh = plsc.VectorSubcoreMesh(core_axis_name="c", subcore_axis_name="s")
    k = functools.partial(
        pl.kernel,
        mesh=mesh,
        compiler_params=pltpu.CompilerParams(needs_layout_passes=False),
        out_type=jax.ShapeDtypeStruct((TOK, D), jnp.float32),
        scratch_types=[
            pltpu.VMEM((VOCAB, D), jnp.float32),
            pltpu.VMEM((CHUNK, D), jnp.float32),
            pltpu.VMEM((CHUNK, D), jnp.float32),
            pltpu.VMEM((CHUNK, D), jnp.float32),
            pltpu.VMEM((D,), jnp.float32),
            pltpu.VMEM((D,), jnp.float32),
            pltpu.VMEM((TPW,), jnp.int32),
            pltpu.VMEM((NCHUNK, CHUNK), jnp.int32),
            pltpu.VMEM((TPW,), jnp.float32),
            pltpu.VMEM((TPW,), jnp.float32),
            pltpu.SemaphoreType.DMA,
            pltpu.SemaphoreType.DMA,
            pltpu.SemaphoreType.DMA,
            pltpu.SemaphoreType.DMA,
            pltpu.SemaphoreType.DMA,
            pltpu.SemaphoreType.DMA,
        ],
    )(_sc_body)
    return k(ids_f, pos2, wsc_f, am_f, word_emb, pos_emb, g, b)


def kernel(input_ids, attention_mask, word_emb, pos_emb, ln_gamma, ln_beta):
    ids = input_ids.astype(jnp.int32)
    attn = attention_mask.astype(jnp.float32)
    pos_ids, wscale = _prep(ids, attn)
    out = _sc_embed(
        ids.reshape(TOK), pos_ids.reshape(NW * NCHUNK, CHUNK),
        wscale.reshape(TOK), attn.reshape(TOK),
        word_emb, pos_emb, ln_gamma, ln_beta)
    return out.reshape(B, S, D)


# speedup vs baseline: 4.4091x; 2.4280x over previous
import functools

import jax
import jax.numpy as jnp
from jax import lax
from jax.experimental import pallas as pl
from jax.experimental.pallas import tpu as pltpu
from jax.experimental.pallas import tpu_sc as plsc

PAD_IDX = 1
MASK_ID = 32
LN_EPS = 1e-05
B, S, D = 4, 2048, 2048
VOCAB, MAX_POS = 33, 4096

NC, NS = 2, 16
NW = NC * NS
TOK = B * S
TPW = TOK // NW
CHUNK = 4
NCHUNK = TPW // CHUNK
NVREG = D // 16


def _prep_body(ids_ref, attn_ref, pos_ref, wsc_ref):
    ids = ids_ref[...]
    attn = attn_ref[...]
    nonpad = (ids != PAD_IDX).astype(jnp.int32)
    c = nonpad
    sh = 1
    while sh < S:
        c = c + jnp.concatenate(
            [jnp.zeros((B, sh), jnp.int32), c[:, : S - sh]], axis=1)
        sh *= 2
    pos_ref[...] = c * nonpad + PAD_IDX
    is_mask = ids == MASK_ID
    n_mask = jnp.sum(is_mask.astype(jnp.float32), axis=1, keepdims=True)
    src = jnp.sum(attn, axis=1, keepdims=True)
    scale = (1.0 - 0.15 * 0.8) / (1.0 - n_mask / src)
    wsc_ref[...] = jnp.where(is_mask, 0.0, jnp.broadcast_to(scale, (B, S)))


def _prep(ids, attn):
    return pl.pallas_call(
        _prep_body,
        out_shape=[
            jax.ShapeDtypeStruct((B, S), jnp.int32),
            jax.ShapeDtypeStruct((B, S), jnp.float32),
        ],
    )(ids, attn)


def _sc_body(ids_hbm, pos2_hbm, wsc_hbm, am_hbm, wtab_hbm, ptab_hbm,
             g_hbm, b_hbm, out_hbm,
             wtab_v, pr0_v, pr1_v, pr2_v, g_v, b_v,
             ids_v, pos_v, wsc_v, am_v,
             gs0, gs1, gs2, ws0, ws1, ws2):
    wid = lax.axis_index("s") * NC + lax.axis_index("c")
    base = wid * TPW

    pltpu.sync_copy(ids_hbm.at[pl.ds(base, TPW)], ids_v)
    pltpu.sync_copy(pos2_hbm.at[pl.ds(wid * NCHUNK, NCHUNK)], pos_v)
    pltpu.sync_copy(wsc_hbm.at[pl.ds(base, TPW)], wsc_v)
    pltpu.sync_copy(am_hbm.at[pl.ds(base, TPW)], am_v)
    pltpu.sync_copy(wtab_hbm, wtab_v)
    pltpu.sync_copy(g_hbm, g_v)
    pltpu.sync_copy(b_hbm, b_v)

    iota16 = lax.iota(jnp.int32, 16)
    zeros16 = jnp.zeros((16,), jnp.int32)
    inv_d = 1.0 / D

    zf = jnp.zeros((16,), jnp.float32)
    bufs = (pr0_v, pr1_v, pr2_v)
    gsems = (gs0, gs1, gs2)
    wsems = (ws0, ws1, ws2)

    def gather_start(c, i):
        pltpu.make_async_copy(ptab_hbm.at[pos_v.at[c]], bufs[i],
                              gsems[i]).start()

    def gather_wait(c, i):
        pltpu.make_async_copy(ptab_hbm.at[pos_v.at[c]], bufs[i],
                              gsems[i]).wait()

    def write_start(c, i):
        pltpu.make_async_copy(bufs[i],
                              out_hbm.at[pl.ds(base + c * CHUNK, CHUNK)],
                              wsems[i]).start()

    def write_wait(c, i):
        pltpu.make_async_copy(bufs[i],
                              out_hbm.at[pl.ds(base + c * CHUNK, CHUNK)],
                              wsems[i]).wait()

    def compute_chunk(c, i, issue_next):
        prows_v = bufs[i]
        gather_wait(c, i)
        tok0 = c * CHUNK

        row_splats, wscvs, amvs = [], [], []
        for t in range(CHUNK):
            tok_splat = zeros16 + (tok0 + t)
            row_splats.append(plsc.load_gather(ids_v, [tok_splat]))
            wscvs.append(plsc.load_gather(wsc_v, [tok_splat]))
            amvs.append(plsc.load_gather(am_v, [tok_splat]))

        @plsc.parallel_loop(0, NVREG, 1, unroll=4, carry=(zf,) * (2 * CHUNK))
        def p1(j, acc):
            cols = iota16 + j * 16
            sl = pl.ds(j * 16, 16)
            new = []
            for t in range(CHUNK):
                w = plsc.load_gather(wtab_v, [row_splats[t], cols])
                x = w * wscvs[t] + prows_v[t, sl]
                prows_v[t, sl] = x
                new.append(acc[2 * t] + x)
                new.append(acc[2 * t + 1] + x * x)
            return tuple(new)

        acc = p1

        a1s, a0s = [], []
        for t in range(CHUNK):
            mu = jnp.sum(acc[2 * t]) * inv_d
            var = jnp.sum(acc[2 * t + 1]) * inv_d - mu * mu
            vv = jnp.broadcast_to(var + LN_EPS, (16,))
            yi = jnp.int32(0x5F3759DF) - (
                plsc.bitcast(vv, jnp.int32) >> jnp.int32(1))
            y = plsc.bitcast(yi, jnp.float32)
            for _ in range(3):
                y = y * (1.5 - 0.5 * vv * y * y)
            a1s.append(y * amvs[t])
            a0s.append((-mu) * y * amvs[t])

        @plsc.parallel_loop(0, NVREG, 1, unroll=4)
        def p2(j):
            sl = pl.ds(j * 16, 16)
            g = g_v[sl]
            b = b_v[sl]
            for t in range(CHUNK):
                x = prows_v[t, sl]
                prows_v[t, sl] = g * (x * a1s[t] + a0s[t]) + b * amvs[t]

        if issue_next:
            nb = (i + 2) % 3

            @pl.when(c >= 1)
            def _():
                write_wait(c - 1, nb)

            gather_start(c + 2, nb)
        write_start(c, i)

    gather_start(0, 0)
    gather_start(1, 1)

    def ring_body(g, carry):
        c0 = g * 3
        compute_chunk(c0, 0, True)
        compute_chunk(c0 + 1, 1, True)
        compute_chunk(c0 + 2, 2, True)
        return carry

    lax.fori_loop(0, (NCHUNK - 4) // 3, ring_body, 0)

    compute_chunk(NCHUNK - 4, (NCHUNK - 4) % 3, True)
    compute_chunk(NCHUNK - 3, (NCHUNK - 3) % 3, True)
    compute_chunk(NCHUNK - 2, (NCHUNK - 2) % 3, False)
    compute_chunk(NCHUNK - 1, (NCHUNK - 1) % 3, False)
    write_wait(NCHUNK - 3, (NCHUNK - 3) % 3)
    write_wait(NCHUNK - 2, (NCHUNK - 2) % 3)
    write_wait(NCHUNK - 1, (NCHUNK - 1) % 3)


@functools.partial(jax.jit, static_argnums=())
def _sc_embed(ids_f, pos2, wsc_f, am_f, word_emb, pos_emb, g, b):
    mesh = plsc.VectorSubcoreMesh(core_axis_name="c", subcore_axis_name="s")
    k = functools.partial(
        pl.kernel,
        mesh=mesh,
        compiler_params=pltpu.CompilerParams(needs_layout_passes=False),
        out_type=jax.ShapeDtypeStruct((TOK, D), jnp.float32),
        scratch_types=[
            pltpu.VMEM((VOCAB, D), jnp.float32),
            pltpu.VMEM((CHUNK, D), jnp.float32),
            pltpu.VMEM((CHUNK, D), jnp.float32),
            pltpu.VMEM((CHUNK, D), jnp.float32),
            pltpu.VMEM((D,), jnp.float32),
            pltpu.VMEM((D,), jnp.float32),
            pltpu.VMEM((TPW,), jnp.int32),
            pltpu.VMEM((NCHUNK, CHUNK), jnp.int32),
            pltpu.VMEM((TPW,), jnp.float32),
            pltpu.VMEM((TPW,), jnp.float32),
            pltpu.SemaphoreType.DMA,
            pltpu.SemaphoreType.DMA,
            pltpu.SemaphoreType.DMA,
            pltpu.SemaphoreType.DMA,
            pltpu.SemaphoreType.DMA,
            pltpu.SemaphoreType.DMA,
        ],
    )(_sc_body)
    return k(ids_f, pos2, wsc_f, am_f, word_emb, pos_emb, g, b)


def kernel(input_ids, attention_mask, word_emb, pos_emb, ln_gamma, ln_beta):
    ids = input_ids.astype(jnp.int32)
    attn = attention_mask.astype(jnp.float32)
    pos_ids, wscale = _prep(ids, attn)
    out = _sc_embed(
        ids.reshape(TOK), pos_ids.reshape(NW * NCHUNK, CHUNK),
        wscale.reshape(TOK), attn.reshape(TOK),
        word_emb, pos_emb, ln_gamma, ln_beta)
    return out.reshape(B, S, D)
